# baseline (speedup 1.0000x reference)
import functools
import math

import jax
import jax.numpy as jnp
from jax import lax
from jax.experimental import pallas as pl
from jax.experimental.pallas import tpu as pltpu
from jax.experimental.pallas import tpu_sc as plsc

H = 16
NBUC = 32
QL = 2048
KL = 2048
VW = 4224
NSHIFT = 8
WAVE = 16


def _vals_body(delta_ref, table_t_ref, vals8_ref):
    r = jax.lax.broadcasted_iota(jnp.int32, (H, NSHIFT, VW), 1)
    x = jax.lax.broadcasted_iota(jnp.int32, (H, NSHIFT, VW), 2)
    d = x - r - (QL - 1) + delta_ref[0]
    rb = jnp.where(d > 0, 16, 0).astype(jnp.int32)
    a = jnp.abs(d)
    is_small = a < 8
    rp_safe = jnp.maximum(a, 1)
    large = 8 + (
        jnp.log(rp_safe.astype(jnp.float32) / 8)
        / math.log(128 / 8)
        * (16 - 8)
    ).astype(jnp.int32)
    large = jnp.minimum(large, jnp.full_like(large, 15))
    bucket = rb + jnp.where(is_small, a, large)
    acc = jnp.zeros((H, NSHIFT, VW), jnp.float32)
    for b in range(NBUC):
        acc = jnp.where(bucket == b, table_t_ref[:, pl.ds(b, 1)][:, :, None], acc)
    vals8_ref[...] = acc


def _sc_body(vals8_hbm, out_hbm, vv, sem):
    wid = lax.axis_index("s") * 2 + lax.axis_index("c")
    head = wid // 2
    base = (wid % 2) * (QL // 2)
    pltpu.sync_copy(vals8_hbm.at[pl.ds(head * (NSHIFT * VW), NSHIFT * VW)], vv)

    def fire(t):
        i = base + t
        s = (QL - 1) - i
        r = (i + 1) % NSHIFT
        off = pl.multiple_of(r * VW + s + r, 8)
        dst = pl.multiple_of((head * QL + i) * KL, 8)
        pltpu.async_copy(vv.at[pl.ds(off, KL)], out_hbm.at[pl.ds(dst, KL)], sem)

    def drain_one():
        pltpu.make_async_copy(
            out_hbm.at[pl.ds(0, KL)], vv.at[pl.ds(0, KL)], sem
        ).wait()

    for u in range(WAVE):
        fire(u)

    def step(t, carry):
        fire(t + WAVE)
        drain_one()
        return carry

    lax.fori_loop(0, (QL // 2) - WAVE, step, 0, unroll=False)
    for _ in range(WAVE):
        drain_one()


def kernel(query_length, key_length, relative_attention_bias):
    delta = (
        (jnp.asarray(key_length, jnp.int32) - KL)
        - (jnp.asarray(query_length, jnp.int32) - QL)
    ).reshape(1)
    table_t = relative_attention_bias.T
    vals8 = pl.pallas_call(
        _vals_body,
        in_specs=[
            pl.BlockSpec(memory_space=pltpu.SMEM),
            pl.BlockSpec((H, NBUC), lambda: (0, 0)),
        ],
        out_specs=pl.BlockSpec((H, NSHIFT, VW), lambda: (0, 0, 0)),
        out_shape=jax.ShapeDtypeStruct((H, NSHIFT, VW), jnp.float32),
    )(delta, table_t)

    sc_call = functools.partial(
        pl.kernel,
        out_type=jax.ShapeDtypeStruct((H * QL * KL,), jnp.float32),
        mesh=plsc.VectorSubcoreMesh(core_axis_name="c", subcore_axis_name="s"),
        scratch_types=[
            pltpu.VMEM((NSHIFT * VW,), jnp.float32),
            pltpu.SemaphoreType.DMA,
        ],
    )(_sc_body)
    out = sc_call(vals8.reshape(H * NSHIFT * VW))
    return out.reshape(1, H, QL, KL)
